# baseline (speedup 1.0000x reference)
import functools

import jax
import jax.numpy as jnp
from jax.experimental import pallas as pl
from jax.experimental.pallas import tpu as pltpu

USER_N = 6000
ITEM_N = 4000
NODES = USER_N + ITEM_N
D = 128
RIS_ADJ_LAM = 0.2
RIS_LAM = 0.5


def _lrelu(x):
    return jnp.where(x >= 0, x, 0.2 * x)


def _feats_body(x_ref, w_ref, o_ref):
    y = jnp.dot(x_ref[...], w_ref[...], preferred_element_type=jnp.float32)
    y = _lrelu(y)
    n = jnp.sqrt(jnp.sum(y * y, axis=1, keepdims=True))
    o_ref[...] = y / jnp.maximum(n, 1e-12)


def _feats(x, w, blk=400):
    m, k = x.shape
    return pl.pallas_call(
        _feats_body,
        grid=(m // blk,),
        in_specs=[pl.BlockSpec((blk, k), lambda i: (i, 0)),
                  pl.BlockSpec((k, D), lambda i: (0, 0))],
        out_specs=pl.BlockSpec((blk, D), lambda i: (i, 0)),
        out_shape=jax.ShapeDtypeStruct((m, D), jnp.float32),
    )(x, w)


def _spmm(idx, vals, x):
    gathered = x[idx[1]] * vals[:, None]
    return jax.ops.segment_sum(gathered, idx[0], num_segments=NODES)


def _combine_body(mw_ref, imgbase_ref, s1_ref, s2_ref, s3_ref, s4_ref,
                  s5_ref, o_ref):
    w = jax.nn.softmax(mw_ref[0])
    ei = imgbase_ref[...] + s2_ref[...] + RIS_ADJ_LAM * s1_ref[...]
    et = s4_ref[...] + s5_ref[...] + RIS_ADJ_LAM * s3_ref[...]
    o_ref[...] = w[0] * ei + w[1] * et


def _combine(mw, imgbase, s1, s2, s3, s4, s5, blk=2000):
    specs = [pl.BlockSpec((1, 2), lambda i: (0, 0))]
    specs += [pl.BlockSpec((blk, D), lambda i: (i, 0))] * 6
    return pl.pallas_call(
        _combine_body,
        grid=(NODES // blk,),
        in_specs=specs,
        out_specs=pl.BlockSpec((blk, D), lambda i: (i, 0)),
        out_shape=jax.ShapeDtypeStruct((NODES, D), jnp.float32),
    )(mw.reshape(1, 2), imgbase, s1, s2, s3, s4, s5)


def _attn_body(e_ref, w_ref, o_ref):
    e = e_ref[...]
    s = jnp.dot(e, w_ref[...], preferred_element_type=jnp.float32)
    s = jax.nn.softmax(s, axis=0)
    o_ref[...] = _lrelu(e * s)


def _attn(e, w):
    return pl.pallas_call(
        _attn_body,
        in_specs=[pl.BlockSpec((NODES, D), lambda: (0, 0)),
                  pl.BlockSpec((D, 1), lambda: (0, 0))],
        out_specs=pl.BlockSpec((NODES, D), lambda: (0, 0)),
        out_shape=jax.ShapeDtypeStruct((NODES, D), jnp.float32),
    )(e, w)


def _final_body(e_ref, w_ref, modal_ref, g1_ref, o_ref):
    e = e_ref[...]
    s = jnp.dot(e, w_ref[...], preferred_element_type=jnp.float32)
    s = jax.nn.softmax(s, axis=0)
    g2 = _lrelu(e * s)
    m = modal_ref[...]
    n = jnp.sqrt(jnp.sum(m * m, axis=1, keepdims=True))
    o_ref[...] = m + g1_ref[...] + g2 + RIS_LAM * (m / jnp.maximum(n, 1e-12))


def _final(e, w, modal, g1):
    return pl.pallas_call(
        _final_body,
        in_specs=[pl.BlockSpec((NODES, D), lambda: (0, 0)),
                  pl.BlockSpec((D, 1), lambda: (0, 0)),
                  pl.BlockSpec((NODES, D), lambda: (0, 0)),
                  pl.BlockSpec((NODES, D), lambda: (0, 0))],
        out_specs=pl.BlockSpec((NODES, D), lambda: (0, 0)),
        out_shape=jax.ShapeDtypeStruct((NODES, D), jnp.float32),
    )(e, w, modal, g1)


def kernel(adj_idx, adj_vals, image_adj_idx, image_adj_vals, text_adj_idx,
           text_adj_vals, image_embedding, text_embedding, uEmbeds, iEmbeds,
           image_trans, text_trans, modal_weight, att_w0, att_w1):
    img_n = _feats(image_embedding, image_trans)
    txt_n = _feats(text_embedding, text_trans)

    base = jnp.concatenate([uEmbeds, iEmbeds], axis=0)
    x4 = jnp.concatenate([uEmbeds, txt_n], axis=0)

    s1 = _spmm(image_adj_idx, image_adj_vals, base)
    s2 = _spmm(adj_idx, adj_vals, base)
    s3 = _spmm(text_adj_idx, text_adj_vals, base)
    s4 = _spmm(adj_idx, adj_vals, x4)

    x5 = jnp.concatenate([s4[:USER_N], iEmbeds], axis=0)
    s5 = _spmm(adj_idx, adj_vals, x5)

    imgbase = jnp.concatenate([uEmbeds, img_n], axis=0)
    modal = _combine(modal_weight, imgbase, s1, s2, s3, s4, s5)

    e1 = _spmm(adj_idx, adj_vals, modal)
    g1 = _attn(e1, att_w0)
    e2 = _spmm(adj_idx, adj_vals, g1)
    out = _final(e2, att_w1, modal, g1)
    return (out[:USER_N], out[USER_N:])


# speedup vs baseline: 3.2953x; 3.2953x over previous
import functools

import jax
import jax.numpy as jnp
from jax import lax
from jax.experimental import pallas as pl
from jax.experimental.pallas import tpu as pltpu
from jax.experimental.pallas import tpu_sc as plsc

USER_N = 6000
ITEM_N = 4000
NODES = USER_N + ITEM_N
D = 128
E_EDGES = 320000
RIS_ADJ_LAM = 0.2
RIS_LAM = 0.5

_C = 128
_NCHUNKS = E_EDGES // _C
_NSUB = 16
_STRIPE = 200
_NSTRIPES = NODES // _STRIPE

_MESH = plsc.VectorSubcoreMesh(
    core_axis_name="c", subcore_axis_name="s", num_cores=2, num_subcores=_NSUB)


def _lrelu(x):
    return jnp.where(x >= 0, x, 0.2 * x)


def _sc_zero_acc(sid, acc, blk_v):
    @pl.loop(0, _STRIPE)
    def _zero_rows(r):
        for q in range(8):
            blk_v[r, pl.ds(q * 16, 16)] = jnp.zeros((16,), jnp.float32)

    my_n = (_NSTRIPES - sid + _NSUB - 1) // _NSUB

    @pl.loop(0, my_n)
    def _blast(i):
        row = pl.multiple_of((sid + i * _NSUB) * _STRIPE, 8)
        pltpu.sync_copy(blk_v, acc.at[pl.ds(row, _STRIPE), :])


def _sc_edge_chunk(base, src_hbm, dst_hbm, vals_hbm, x_hbm, acc,
                   src_v, dst_v, vals_v, rows_v, sem):
    pltpu.sync_copy(src_hbm.at[pl.ds(base, _C)], src_v)
    pltpu.sync_copy(dst_hbm.at[pl.ds(base, _C)], dst_v)
    pltpu.sync_copy(vals_hbm.at[pl.ds(base, _C)], vals_v)
    pltpu.async_copy(x_hbm.at[src_v], rows_v, sem).wait()

    @pl.loop(0, _C)
    def _scale(e):
        b = plsc.load_gather(vals_v, [jnp.full((16,), e, jnp.int32)])
        for q in range(8):
            sl = pl.ds(q * 16, 16)
            rows_v[e, sl] = rows_v[e, sl] * b

    pltpu.sync_copy(rows_v, acc.at[dst_v], add=True)


def _sc_dump(sid, acc, blk_v, out_slice):
    my_n = (_NSTRIPES - sid + _NSUB - 1) // _NSUB

    @pl.loop(0, my_n)
    def _dump(i):
        row = pl.multiple_of((sid + i * _NSUB) * _STRIPE, 8)
        pltpu.sync_copy(acc.at[pl.ds(row, _STRIPE), :], blk_v)
        pltpu.sync_copy(blk_v, out_slice.at[pl.ds(row, _STRIPE), :])


def _sc_run_task(sid, chunk0, nchunks, src_hbm, dst_hbm, vals_hbm, x_hbm,
                 out_slice, acc, src_v, dst_v, vals_v, rows_v, blk_v, sem):
    _sc_zero_acc(sid, acc, blk_v)
    plsc.subcore_barrier()

    my_n = (nchunks - sid + _NSUB - 1) // _NSUB

    @pl.loop(0, my_n)
    def _edges(i):
        k = chunk0 + sid + i * _NSUB
        _sc_edge_chunk(k * _C, src_hbm, dst_hbm, vals_hbm, x_hbm, acc,
                       src_v, dst_v, vals_v, rows_v, sem)

    plsc.subcore_barrier()
    _sc_dump(sid, acc, blk_v, out_slice)
    plsc.subcore_barrier()


def _spmm4_body(img_s, img_d, img_v, adj_s, adj_d, adj_v, txt_s, txt_d, txt_v,
                base_x, x4, out_hbm,
                acc, src_v, dst_v, vals_v, rows_v, blk_v, sem):
    cid = lax.axis_index("c")
    sid = lax.axis_index("s")
    task_sets = [
        [(img_s, img_d, img_v, base_x, 0), (adj_s, adj_d, adj_v, base_x, 1)],
        [(txt_s, txt_d, txt_v, base_x, 2), (adj_s, adj_d, adj_v, x4, 3)],
    ]
    for core, tasks in enumerate(task_sets):
        @pl.when(cid == core)
        def _run(tasks=tasks):
            for (s, d, v, x, slot) in tasks:
                _sc_run_task(sid, 0, _NCHUNKS, s, d, v, x, out_hbm.at[slot],
                             acc, src_v, dst_v, vals_v, rows_v, blk_v, sem)


def _spmm1_body(src, dst, vals, x, out_hbm,
                acc, src_v, dst_v, vals_v, rows_v, blk_v, sem):
    cid = lax.axis_index("c")
    sid = lax.axis_index("s")
    half = _NCHUNKS // 2
    _sc_run_task(sid, cid * half, half, src, dst, vals, x, out_hbm.at[cid],
                 acc, src_v, dst_v, vals_v, rows_v, blk_v, sem)


_SC_SCRATCH = [
    pltpu.VMEM_SHARED((NODES, D), jnp.float32),
    pltpu.VMEM((_C,), jnp.int32),
    pltpu.VMEM((_C,), jnp.int32),
    pltpu.VMEM((_C,), jnp.float32),
    pltpu.VMEM((_C, D), jnp.float32),
    pltpu.VMEM((_STRIPE, D), jnp.float32),
    pltpu.SemaphoreType.DMA,
]

_SC_PARAMS = pltpu.CompilerParams(needs_layout_passes=False)

_spmm4_call = pl.kernel(
    _spmm4_body,
    out_type=jax.ShapeDtypeStruct((4, NODES, D), jnp.float32),
    mesh=_MESH,
    scratch_types=_SC_SCRATCH,
    compiler_params=_SC_PARAMS,
)

_spmm1_call = pl.kernel(
    _spmm1_body,
    out_type=jax.ShapeDtypeStruct((2, NODES, D), jnp.float32),
    mesh=_MESH,
    scratch_types=_SC_SCRATCH,
    compiler_params=_SC_PARAMS,
)


def _spmm4(img_idx, img_vals, adj_idx, adj_vals, txt_idx, txt_vals, base_x, x4):
    return _spmm4_call(img_idx[1], img_idx[0], img_vals,
                       adj_idx[1], adj_idx[0], adj_vals,
                       txt_idx[1], txt_idx[0], txt_vals, base_x, x4)


def _spmm1(idx, vals, x):
    return _spmm1_call(idx[1], idx[0], vals, x)


def _feats_body(x_ref, w_ref, o_ref):
    y = jnp.dot(x_ref[...], w_ref[...], preferred_element_type=jnp.float32)
    y = _lrelu(y)
    n = jnp.sqrt(jnp.sum(y * y, axis=1, keepdims=True))
    o_ref[...] = y / jnp.maximum(n, 1e-12)


def _feats(x, w, blk=400):
    m, k = x.shape
    return pl.pallas_call(
        _feats_body,
        grid=(m // blk,),
        in_specs=[pl.BlockSpec((blk, k), lambda i: (i, 0)),
                  pl.BlockSpec((k, D), lambda i: (0, 0))],
        out_specs=pl.BlockSpec((blk, D), lambda i: (i, 0)),
        out_shape=jax.ShapeDtypeStruct((m, D), jnp.float32),
    )(x, w)


def _combine_body(mw_ref, imgbase_ref, s1_ref, s2_ref, s3_ref, s4_ref,
                  s5a_ref, s5b_ref, o_ref):
    w = jax.nn.softmax(mw_ref[0])
    ei = imgbase_ref[...] + s2_ref[...] + RIS_ADJ_LAM * s1_ref[...]
    et = s4_ref[...] + s5a_ref[...] + s5b_ref[...] + RIS_ADJ_LAM * s3_ref[...]
    o_ref[...] = w[0] * ei + w[1] * et


def _combine(mw, imgbase, s1, s2, s3, s4, s5a, s5b, blk=2000):
    specs = [pl.BlockSpec((1, 2), lambda i: (0, 0))]
    specs += [pl.BlockSpec((blk, D), lambda i: (i, 0))] * 7
    return pl.pallas_call(
        _combine_body,
        grid=(NODES // blk,),
        in_specs=specs,
        out_specs=pl.BlockSpec((blk, D), lambda i: (i, 0)),
        out_shape=jax.ShapeDtypeStruct((NODES, D), jnp.float32),
    )(mw.reshape(1, 2), imgbase, s1, s2, s3, s4, s5a, s5b)


def _attn_body(ea_ref, eb_ref, w_ref, o_ref):
    e = ea_ref[...] + eb_ref[...]
    s = jnp.dot(e, w_ref[...], preferred_element_type=jnp.float32)
    s = jax.nn.softmax(s, axis=0)
    o_ref[...] = _lrelu(e * s)


def _attn(ea, eb, w):
    return pl.pallas_call(
        _attn_body,
        in_specs=[pl.BlockSpec((NODES, D), lambda: (0, 0)),
                  pl.BlockSpec((NODES, D), lambda: (0, 0)),
                  pl.BlockSpec((D, 1), lambda: (0, 0))],
        out_specs=pl.BlockSpec((NODES, D), lambda: (0, 0)),
        out_shape=jax.ShapeDtypeStruct((NODES, D), jnp.float32),
    )(ea, eb, w)


def _final_body(ea_ref, eb_ref, w_ref, modal_ref, g1_ref, o_ref):
    e = ea_ref[...] + eb_ref[...]
    s = jnp.dot(e, w_ref[...], preferred_element_type=jnp.float32)
    s = jax.nn.softmax(s, axis=0)
    g2 = _lrelu(e * s)
    m = modal_ref[...]
    n = jnp.sqrt(jnp.sum(m * m, axis=1, keepdims=True))
    o_ref[...] = m + g1_ref[...] + g2 + RIS_LAM * (m / jnp.maximum(n, 1e-12))


def _final(ea, eb, w, modal, g1):
    return pl.pallas_call(
        _final_body,
        in_specs=[pl.BlockSpec((NODES, D), lambda: (0, 0)),
                  pl.BlockSpec((NODES, D), lambda: (0, 0)),
                  pl.BlockSpec((D, 1), lambda: (0, 0)),
                  pl.BlockSpec((NODES, D), lambda: (0, 0)),
                  pl.BlockSpec((NODES, D), lambda: (0, 0))],
        out_specs=pl.BlockSpec((NODES, D), lambda: (0, 0)),
        out_shape=jax.ShapeDtypeStruct((NODES, D), jnp.float32),
    )(ea, eb, w, modal, g1)


def kernel(adj_idx, adj_vals, image_adj_idx, image_adj_vals, text_adj_idx,
           text_adj_vals, image_embedding, text_embedding, uEmbeds, iEmbeds,
           image_trans, text_trans, modal_weight, att_w0, att_w1):
    img_n = _feats(image_embedding, image_trans)
    txt_n = _feats(text_embedding, text_trans)

    base = jnp.concatenate([uEmbeds, iEmbeds], axis=0)
    x4 = jnp.concatenate([uEmbeds, txt_n], axis=0)

    s14 = _spmm4(image_adj_idx, image_adj_vals, adj_idx, adj_vals,
                 text_adj_idx, text_adj_vals, base, x4)
    s1, s2, s3, s4 = s14[0], s14[1], s14[2], s14[3]

    x5 = jnp.concatenate([s4[:USER_N], iEmbeds], axis=0)
    s5 = _spmm1(adj_idx, adj_vals, x5)

    imgbase = jnp.concatenate([uEmbeds, img_n], axis=0)
    modal = _combine(modal_weight, imgbase, s1, s2, s3, s4, s5[0], s5[1])

    e1 = _spmm1(adj_idx, adj_vals, modal)
    g1 = _attn(e1[0], e1[1], att_w0)
    e2 = _spmm1(adj_idx, adj_vals, g1)
    out = _final(e2[0], e2[1], att_w1, modal, g1)
    return (out[:USER_N], out[USER_N:])


# speedup vs baseline: 7.3329x; 2.2253x over previous
import functools

import jax
import jax.numpy as jnp
from jax import lax
from jax.experimental import pallas as pl
from jax.experimental.pallas import tpu as pltpu
from jax.experimental.pallas import tpu_sc as plsc

USER_N = 6000
ITEM_N = 4000
NODES = USER_N + ITEM_N
D = 128
E_EDGES = 320000
RIS_ADJ_LAM = 0.2
RIS_LAM = 0.5

_C = 128
_NCHUNKS = E_EDGES // _C
_NSUB = 16
_STRIPE = 80
_NSTRIPES = NODES // _STRIPE

_MESH = plsc.VectorSubcoreMesh(
    core_axis_name="c", subcore_axis_name="s", num_cores=2, num_subcores=_NSUB)


def _lrelu(x):
    return jnp.where(x >= 0, x, 0.2 * x)


def _sc_zero_acc(sid, acc, zrows):
    @pl.loop(0, _STRIPE)
    def _zero_rows(r):
        for q in range(8):
            zrows[r, pl.ds(q * 16, 16)] = jnp.zeros((16,), jnp.float32)

    my_n = (_NSTRIPES - sid + _NSUB - 1) // _NSUB

    @pl.loop(0, my_n)
    def _blast(i):
        row = pl.multiple_of((sid + i * _NSUB) * _STRIPE, 8)
        pltpu.sync_copy(zrows.at[pl.ds(0, _STRIPE), :],
                        acc.at[pl.ds(row, _STRIPE), :])


def _sc_edge_loop(sid, chunk0, nchunks, src_hbm, dst_hbm, vals_hbm, x_hbm, acc,
                  srcb, dstb, valsb, rowsb, gsem, isems):
    n = (nchunks - sid + _NSUB - 1) // _NSUB

    def cbase(i):
        return (chunk0 + sid + i * _NSUB) * _C

    def fetch_idx(i, b):
        base = cbase(i)
        pltpu.async_copy(src_hbm.at[pl.ds(base, _C)], srcb[b], isems[b])
        pltpu.async_copy(dst_hbm.at[pl.ds(base, _C)], dstb[b], isems[b])
        pltpu.async_copy(vals_hbm.at[pl.ds(base, _C)], valsb[b], isems[b])

    def wait_idx(b):
        pltpu.make_async_copy(src_hbm.at[pl.ds(0, _C)], srcb[b], isems[b]).wait()
        pltpu.make_async_copy(dst_hbm.at[pl.ds(0, _C)], dstb[b], isems[b]).wait()
        pltpu.make_async_copy(vals_hbm.at[pl.ds(0, _C)], valsb[b], isems[b]).wait()

    def start_gather(b):
        pltpu.async_copy(x_hbm.at[srcb[b]], rowsb[b], gsem)

    def wait_gather(b):
        pltpu.make_async_copy(x_hbm.at[pl.ds(0, _C), :], rowsb[b], gsem).wait()

    fetch_idx(0, 0)
    fetch_idx(1, 1)
    wait_idx(0)
    start_gather(0)

    @pl.loop(0, (n + 1) // 2)
    def _pair(t):
        for b in (0, 1):
            i = 2 * t + b

            @pl.when(i < n)
            def _step(i=i, b=b):
                wait_gather(b)

                @pl.when(i + 1 < n)
                def _next_gather(i=i, b=b):
                    wait_idx(1 - b)
                    start_gather(1 - b)

                rows = rowsb[b]
                vals = valsb[b]

                @pl.loop(0, _C, unroll=4)
                def _scale(e):
                    bv = plsc.load_gather(vals, [jnp.full((16,), e, jnp.int32)])
                    for q in range(8):
                        sl = pl.ds(q * 16, 16)
                        rows[e, sl] = rows[e, sl] * bv

                @pl.when(i + 2 < n)
                def _next_idx(i=i, b=b):
                    fetch_idx(i + 2, b)

                pltpu.sync_copy(rows, acc.at[dstb[b]], add=True)


def _sc_dump(sid, acc, srows, out_slice):
    my_n = (_NSTRIPES - sid + _NSUB - 1) // _NSUB

    @pl.loop(0, my_n)
    def _dump(i):
        row = pl.multiple_of((sid + i * _NSUB) * _STRIPE, 8)
        pltpu.sync_copy(acc.at[pl.ds(row, _STRIPE), :],
                        srows.at[pl.ds(0, _STRIPE), :])
        pltpu.sync_copy(srows.at[pl.ds(0, _STRIPE), :],
                        out_slice.at[pl.ds(row, _STRIPE), :])


def _sc_run_task(sid, chunk0, nchunks, src_hbm, dst_hbm, vals_hbm, x_hbm,
                 out_slice, acc, srcb, dstb, valsb, rowsb, gsem, isems):
    _sc_zero_acc(sid, acc, rowsb[0])
    plsc.subcore_barrier()
    _sc_edge_loop(sid, chunk0, nchunks, src_hbm, dst_hbm, vals_hbm, x_hbm, acc,
                  srcb, dstb, valsb, rowsb, gsem, isems)
    plsc.subcore_barrier()
    _sc_dump(sid, acc, rowsb[0], out_slice)
    plsc.subcore_barrier()


def _spmm4_body(img_s, img_d, img_v, adj_s, adj_d, adj_v, txt_s, txt_d, txt_v,
                base_x, x4, out_hbm,
                acc, src0, src1, dst0, dst1, vals0, vals1, rows0, rows1,
                gsem, isem0, isem1):
    cid = lax.axis_index("c")
    sid = lax.axis_index("s")
    task_sets = [
        [(img_s, img_d, img_v, base_x, 0), (adj_s, adj_d, adj_v, base_x, 1)],
        [(txt_s, txt_d, txt_v, base_x, 2), (adj_s, adj_d, adj_v, x4, 3)],
    ]
    for core, tasks in enumerate(task_sets):
        @pl.when(cid == core)
        def _run(tasks=tasks):
            for (s, d, v, x, slot) in tasks:
                _sc_run_task(sid, 0, _NCHUNKS, s, d, v, x, out_hbm.at[slot],
                             acc, (src0, src1), (dst0, dst1), (vals0, vals1),
                             (rows0, rows1), gsem, (isem0, isem1))


def _spmm1_body(src, dst, vals, x, out_hbm,
                acc, src0, src1, dst0, dst1, vals0, vals1, rows0, rows1,
                gsem, isem0, isem1):
    cid = lax.axis_index("c")
    sid = lax.axis_index("s")
    half = _NCHUNKS // 2
    _sc_run_task(sid, cid * half, half, src, dst, vals, x, out_hbm.at[cid],
                 acc, (src0, src1), (dst0, dst1), (vals0, vals1),
                 (rows0, rows1), gsem, (isem0, isem1))


_SC_SCRATCH = [
    pltpu.VMEM_SHARED((NODES, D), jnp.float32),
    pltpu.VMEM((_C,), jnp.int32),
    pltpu.VMEM((_C,), jnp.int32),
    pltpu.VMEM((_C,), jnp.int32),
    pltpu.VMEM((_C,), jnp.int32),
    pltpu.VMEM((_C,), jnp.float32),
    pltpu.VMEM((_C,), jnp.float32),
    pltpu.VMEM((_C, D), jnp.float32),
    pltpu.VMEM((_C, D), jnp.float32),
    pltpu.SemaphoreType.DMA,
    pltpu.SemaphoreType.DMA,
    pltpu.SemaphoreType.DMA,
]

_SC_PARAMS = pltpu.CompilerParams(needs_layout_passes=False)

_spmm4_call = pl.kernel(
    _spmm4_body,
    out_type=jax.ShapeDtypeStruct((4, NODES, D), jnp.float32),
    mesh=_MESH,
    scratch_types=_SC_SCRATCH,
    compiler_params=_SC_PARAMS,
)

_spmm1_call = pl.kernel(
    _spmm1_body,
    out_type=jax.ShapeDtypeStruct((2, NODES, D), jnp.float32),
    mesh=_MESH,
    scratch_types=_SC_SCRATCH,
    compiler_params=_SC_PARAMS,
)


def _spmm4(img_idx, img_vals, adj_idx, adj_vals, txt_idx, txt_vals, base_x, x4):
    return _spmm4_call(img_idx[1], img_idx[0], img_vals,
                       adj_idx[1], adj_idx[0], adj_vals,
                       txt_idx[1], txt_idx[0], txt_vals, base_x, x4)


def _spmm1(idx, vals, x):
    return _spmm1_call(idx[1], idx[0], vals, x)


def _feats_body(x_ref, w_ref, o_ref):
    y = jnp.dot(x_ref[...], w_ref[...], preferred_element_type=jnp.float32)
    y = _lrelu(y)
    n = jnp.sqrt(jnp.sum(y * y, axis=1, keepdims=True))
    o_ref[...] = y / jnp.maximum(n, 1e-12)


def _feats(x, w, blk=400):
    m, k = x.shape
    return pl.pallas_call(
        _feats_body,
        grid=(m // blk,),
        in_specs=[pl.BlockSpec((blk, k), lambda i: (i, 0)),
                  pl.BlockSpec((k, D), lambda i: (0, 0))],
        out_specs=pl.BlockSpec((blk, D), lambda i: (i, 0)),
        out_shape=jax.ShapeDtypeStruct((m, D), jnp.float32),
    )(x, w)


def _combine_body(mw_ref, imgbase_ref, s1_ref, s2_ref, s3_ref, s4_ref,
                  s5a_ref, s5b_ref, o_ref):
    w = jax.nn.softmax(mw_ref[0])
    ei = imgbase_ref[...] + s2_ref[...] + RIS_ADJ_LAM * s1_ref[...]
    et = s4_ref[...] + s5a_ref[...] + s5b_ref[...] + RIS_ADJ_LAM * s3_ref[...]
    o_ref[...] = w[0] * ei + w[1] * et


def _combine(mw, imgbase, s1, s2, s3, s4, s5a, s5b, blk=2000):
    specs = [pl.BlockSpec((1, 2), lambda i: (0, 0))]
    specs += [pl.BlockSpec((blk, D), lambda i: (i, 0))] * 7
    return pl.pallas_call(
        _combine_body,
        grid=(NODES // blk,),
        in_specs=specs,
        out_specs=pl.BlockSpec((blk, D), lambda i: (i, 0)),
        out_shape=jax.ShapeDtypeStruct((NODES, D), jnp.float32),
    )(mw.reshape(1, 2), imgbase, s1, s2, s3, s4, s5a, s5b)


def _attn_body(ea_ref, eb_ref, w_ref, o_ref):
    e = ea_ref[...] + eb_ref[...]
    s = jnp.dot(e, w_ref[...], preferred_element_type=jnp.float32)
    s = jax.nn.softmax(s, axis=0)
    o_ref[...] = _lrelu(e * s)


def _attn(ea, eb, w):
    return pl.pallas_call(
        _attn_body,
        in_specs=[pl.BlockSpec((NODES, D), lambda: (0, 0)),
                  pl.BlockSpec((NODES, D), lambda: (0, 0)),
                  pl.BlockSpec((D, 1), lambda: (0, 0))],
        out_specs=pl.BlockSpec((NODES, D), lambda: (0, 0)),
        out_shape=jax.ShapeDtypeStruct((NODES, D), jnp.float32),
    )(ea, eb, w)


def _final_body(ea_ref, eb_ref, w_ref, modal_ref, g1_ref, o_ref):
    e = ea_ref[...] + eb_ref[...]
    s = jnp.dot(e, w_ref[...], preferred_element_type=jnp.float32)
    s = jax.nn.softmax(s, axis=0)
    g2 = _lrelu(e * s)
    m = modal_ref[...]
    n = jnp.sqrt(jnp.sum(m * m, axis=1, keepdims=True))
    o_ref[...] = m + g1_ref[...] + g2 + RIS_LAM * (m / jnp.maximum(n, 1e-12))


def _final(ea, eb, w, modal, g1):
    return pl.pallas_call(
        _final_body,
        in_specs=[pl.BlockSpec((NODES, D), lambda: (0, 0)),
                  pl.BlockSpec((NODES, D), lambda: (0, 0)),
                  pl.BlockSpec((D, 1), lambda: (0, 0)),
                  pl.BlockSpec((NODES, D), lambda: (0, 0)),
                  pl.BlockSpec((NODES, D), lambda: (0, 0))],
        out_specs=pl.BlockSpec((NODES, D), lambda: (0, 0)),
        out_shape=jax.ShapeDtypeStruct((NODES, D), jnp.float32),
    )(ea, eb, w, modal, g1)


def kernel(adj_idx, adj_vals, image_adj_idx, image_adj_vals, text_adj_idx,
           text_adj_vals, image_embedding, text_embedding, uEmbeds, iEmbeds,
           image_trans, text_trans, modal_weight, att_w0, att_w1):
    img_n = _feats(image_embedding, image_trans)
    txt_n = _feats(text_embedding, text_trans)

    base = jnp.concatenate([uEmbeds, iEmbeds], axis=0)
    x4 = jnp.concatenate([uEmbeds, txt_n], axis=0)

    s14 = _spmm4(image_adj_idx, image_adj_vals, adj_idx, adj_vals,
                 text_adj_idx, text_adj_vals, base, x4)
    s1, s2, s3, s4 = s14[0], s14[1], s14[2], s14[3]

    x5 = jnp.concatenate([s4[:USER_N], iEmbeds], axis=0)
    s5 = _spmm1(adj_idx, adj_vals, x5)

    imgbase = jnp.concatenate([uEmbeds, img_n], axis=0)
    modal = _combine(modal_weight, imgbase, s1, s2, s3, s4, s5[0], s5[1])

    e1 = _spmm1(adj_idx, adj_vals, modal)
    g1 = _attn(e1[0], e1[1], att_w0)
    e2 = _spmm1(adj_idx, adj_vals, g1)
    out = _final(e2[0], e2[1], att_w1, modal, g1)
    return (out[:USER_N], out[USER_N:])


# speedup vs baseline: 8.6802x; 1.1837x over previous
import functools

import jax
import jax.numpy as jnp
from jax import lax
from jax.experimental import pallas as pl
from jax.experimental.pallas import tpu as pltpu
from jax.experimental.pallas import tpu_sc as plsc

USER_N = 6000
ITEM_N = 4000
NODES = USER_N + ITEM_N
D = 128
E_EDGES = 320000
RIS_ADJ_LAM = 0.2
RIS_LAM = 0.5

_C = 128
_NCHUNKS = E_EDGES // _C
_NSUB = 16
_STRIPE = 80
_NSTRIPES = NODES // _STRIPE

_MESH = plsc.VectorSubcoreMesh(
    core_axis_name="c", subcore_axis_name="s", num_cores=2, num_subcores=_NSUB)


def _lrelu(x):
    return jnp.where(x >= 0, x, 0.2 * x)


def _sc_zero_acc(sid, acc, zrows):
    @pl.loop(0, _STRIPE)
    def _zero_rows(r):
        for q in range(8):
            zrows[r, pl.ds(q * 16, 16)] = jnp.zeros((16,), jnp.float32)

    my_n = (_NSTRIPES - sid + _NSUB - 1) // _NSUB

    @pl.loop(0, my_n)
    def _blast(i):
        row = pl.multiple_of((sid + i * _NSUB) * _STRIPE, 8)
        pltpu.sync_copy(zrows.at[pl.ds(0, _STRIPE), :],
                        acc.at[pl.ds(row, _STRIPE), :])


def _sc_edge_loop(sid, chunk0, nchunks, src_hbm, dst_hbm, vals_hbm, x_hbm, acc,
                  srcb, dstb, dstsb, valsb, rowsb, gsem, isems, ssems):
    n = (nchunks - sid + _NSUB - 1) // _NSUB

    def cbase(i):
        return (chunk0 + sid + i * _NSUB) * _C

    def fetch_idx(i, b):
        base = cbase(i)
        pltpu.async_copy(src_hbm.at[pl.ds(base, _C)], srcb[b], isems[b])
        pltpu.async_copy(dst_hbm.at[pl.ds(base, _C)], dstb[b], isems[b])
        pltpu.async_copy(vals_hbm.at[pl.ds(base, _C)], valsb[b], isems[b])

    def wait_idx(b):
        pltpu.make_async_copy(src_hbm.at[pl.ds(0, _C)], srcb[b], isems[b]).wait()
        pltpu.make_async_copy(dst_hbm.at[pl.ds(0, _C)], dstb[b], isems[b]).wait()
        pltpu.make_async_copy(vals_hbm.at[pl.ds(0, _C)], valsb[b], isems[b]).wait()

    def start_gather(b):
        pltpu.async_copy(x_hbm.at[srcb[b]], rowsb[b], gsem)

    def wait_gather(b):
        pltpu.make_async_copy(x_hbm.at[pl.ds(0, _C), :], rowsb[b], gsem).wait()

    def wait_scatter(b):
        pltpu.make_async_copy(x_hbm.at[pl.ds(0, _C), :], rowsb[b], ssems[b]).wait()

    fetch_idx(0, 0)
    fetch_idx(1, 1)
    wait_idx(0)
    start_gather(0)

    @pl.loop(0, (n + 1) // 2)
    def _pair(t):
        for b in (0, 1):
            i = 2 * t + b

            @pl.when(i < n)
            def _step(i=i, b=b):
                wait_gather(b)

                @pl.when(i + 1 < n)
                def _next_gather(i=i, b=b):
                    @pl.when(i >= 1)
                    def _drain_prev(b=b):
                        wait_scatter(1 - b)
                    wait_idx(1 - b)
                    start_gather(1 - b)

                rows = rowsb[b]
                vals = valsb[b]
                dsts = dstsb[b]

                for k in range(8):
                    sl = pl.ds(k * 16, 16)
                    dsts[sl] = dstb[b][sl]

                _DNUMS = lax.GatherDimensionNumbers(
                    offset_dims=(), collapsed_slice_dims=(0,),
                    start_index_map=(0,))

                @pl.loop(0, 8)
                def _scale(g):
                    vv = vals[pl.ds(g * 16, 16)]
                    for j in range(16):
                        bv = lax.gather(
                            vv, jnp.full((16, 1), j, jnp.int32), _DNUMS, (1,),
                            mode=lax.GatherScatterMode.PROMISE_IN_BOUNDS)
                        e = g * 16 + j
                        for q in range(8):
                            sl = pl.ds(q * 16, 16)
                            rows[e, sl] = rows[e, sl] * bv

                pltpu.async_copy(rows, acc.at[dsts], ssems[b], add=True)

                @pl.when(i + 2 < n)
                def _next_idx(i=i, b=b):
                    fetch_idx(i + 2, b)

    wait_scatter(0)
    wait_scatter(1)


def _sc_dump(sid, acc, srows, out_slice):
    my_n = (_NSTRIPES - sid + _NSUB - 1) // _NSUB

    @pl.loop(0, my_n)
    def _dump(i):
        row = pl.multiple_of((sid + i * _NSUB) * _STRIPE, 8)
        pltpu.sync_copy(acc.at[pl.ds(row, _STRIPE), :],
                        srows.at[pl.ds(0, _STRIPE), :])
        pltpu.sync_copy(srows.at[pl.ds(0, _STRIPE), :],
                        out_slice.at[pl.ds(row, _STRIPE), :])


def _sc_run_task(sid, chunk0, nchunks, src_hbm, dst_hbm, vals_hbm, x_hbm,
                 out_slice, acc, srcb, dstb, dstsb, valsb, rowsb, gsem, isems,
                 ssems):
    _sc_zero_acc(sid, acc, rowsb[0])
    plsc.subcore_barrier()
    _sc_edge_loop(sid, chunk0, nchunks, src_hbm, dst_hbm, vals_hbm, x_hbm, acc,
                  srcb, dstb, dstsb, valsb, rowsb, gsem, isems, ssems)
    plsc.subcore_barrier()
    _sc_dump(sid, acc, rowsb[0], out_slice)
    plsc.subcore_barrier()


def _spmm4_body(img_s, img_d, img_v, adj_s, adj_d, adj_v, txt_s, txt_d, txt_v,
                base_x, x4, out_hbm,
                acc, src0, src1, dst0, dst1, dsts0, dsts1, vals0, vals1,
                rows0, rows1, gsem, isem0, isem1, ssem0, ssem1):
    cid = lax.axis_index("c")
    sid = lax.axis_index("s")
    task_sets = [
        [(img_s, img_d, img_v, base_x, 0), (adj_s, adj_d, adj_v, base_x, 1)],
        [(txt_s, txt_d, txt_v, base_x, 2), (adj_s, adj_d, adj_v, x4, 3)],
    ]
    for core, tasks in enumerate(task_sets):
        @pl.when(cid == core)
        def _run(tasks=tasks):
            for (s, d, v, x, slot) in tasks:
                _sc_run_task(sid, 0, _NCHUNKS, s, d, v, x, out_hbm.at[slot],
                             acc, (src0, src1), (dst0, dst1), (dsts0, dsts1),
                             (vals0, vals1), (rows0, rows1), gsem,
                             (isem0, isem1), (ssem0, ssem1))


def _spmm1_body(src, dst, vals, x, out_hbm,
                acc, src0, src1, dst0, dst1, dsts0, dsts1, vals0, vals1,
                rows0, rows1, gsem, isem0, isem1, ssem0, ssem1):
    cid = lax.axis_index("c")
    sid = lax.axis_index("s")
    half = _NCHUNKS // 2
    _sc_run_task(sid, cid * half, half, src, dst, vals, x, out_hbm.at[cid],
                 acc, (src0, src1), (dst0, dst1), (dsts0, dsts1),
                 (vals0, vals1), (rows0, rows1), gsem,
                 (isem0, isem1), (ssem0, ssem1))


_SC_SCRATCH = [
    pltpu.VMEM_SHARED((NODES, D), jnp.float32),
    pltpu.VMEM((_C,), jnp.int32),
    pltpu.VMEM((_C,), jnp.int32),
    pltpu.VMEM((_C,), jnp.int32),
    pltpu.VMEM((_C,), jnp.int32),
    pltpu.VMEM((_C,), jnp.int32),
    pltpu.VMEM((_C,), jnp.int32),
    pltpu.VMEM((_C,), jnp.float32),
    pltpu.VMEM((_C,), jnp.float32),
    pltpu.VMEM((_C, D), jnp.float32),
    pltpu.VMEM((_C, D), jnp.float32),
    pltpu.SemaphoreType.DMA,
    pltpu.SemaphoreType.DMA,
    pltpu.SemaphoreType.DMA,
    pltpu.SemaphoreType.DMA,
    pltpu.SemaphoreType.DMA,
]

_SC_PARAMS = pltpu.CompilerParams(needs_layout_passes=False)

_spmm4_call = pl.kernel(
    _spmm4_body,
    out_type=jax.ShapeDtypeStruct((4, NODES, D), jnp.float32),
    mesh=_MESH,
    scratch_types=_SC_SCRATCH,
    compiler_params=_SC_PARAMS,
)

_spmm1_call = pl.kernel(
    _spmm1_body,
    out_type=jax.ShapeDtypeStruct((2, NODES, D), jnp.float32),
    mesh=_MESH,
    scratch_types=_SC_SCRATCH,
    compiler_params=_SC_PARAMS,
)


def _spmm4(img_idx, img_vals, adj_idx, adj_vals, txt_idx, txt_vals, base_x, x4):
    return _spmm4_call(img_idx[1], img_idx[0], img_vals,
                       adj_idx[1], adj_idx[0], adj_vals,
                       txt_idx[1], txt_idx[0], txt_vals, base_x, x4)


def _spmm1(idx, vals, x):
    return _spmm1_call(idx[1], idx[0], vals, x)


def _feats_body(x_ref, w_ref, o_ref):
    y = jnp.dot(x_ref[...], w_ref[...], preferred_element_type=jnp.float32)
    y = _lrelu(y)
    n = jnp.sqrt(jnp.sum(y * y, axis=1, keepdims=True))
    o_ref[...] = y / jnp.maximum(n, 1e-12)


def _feats(x, w, blk=400):
    m, k = x.shape
    return pl.pallas_call(
        _feats_body,
        grid=(m // blk,),
        in_specs=[pl.BlockSpec((blk, k), lambda i: (i, 0)),
                  pl.BlockSpec((k, D), lambda i: (0, 0))],
        out_specs=pl.BlockSpec((blk, D), lambda i: (i, 0)),
        out_shape=jax.ShapeDtypeStruct((m, D), jnp.float32),
    )(x, w)


def _combine_body(mw_ref, imgbase_ref, s1_ref, s2_ref, s3_ref, s4_ref,
                  s5a_ref, s5b_ref, o_ref):
    w = jax.nn.softmax(mw_ref[0])
    ei = imgbase_ref[...] + s2_ref[...] + RIS_ADJ_LAM * s1_ref[...]
    et = s4_ref[...] + s5a_ref[...] + s5b_ref[...] + RIS_ADJ_LAM * s3_ref[...]
    o_ref[...] = w[0] * ei + w[1] * et


def _combine(mw, imgbase, s1, s2, s3, s4, s5a, s5b, blk=2000):
    specs = [pl.BlockSpec((1, 2), lambda i: (0, 0))]
    specs += [pl.BlockSpec((blk, D), lambda i: (i, 0))] * 7
    return pl.pallas_call(
        _combine_body,
        grid=(NODES // blk,),
        in_specs=specs,
        out_specs=pl.BlockSpec((blk, D), lambda i: (i, 0)),
        out_shape=jax.ShapeDtypeStruct((NODES, D), jnp.float32),
    )(mw.reshape(1, 2), imgbase, s1, s2, s3, s4, s5a, s5b)


def _attn_body(ea_ref, eb_ref, w_ref, o_ref):
    e = ea_ref[...] + eb_ref[...]
    s = jnp.dot(e, w_ref[...], preferred_element_type=jnp.float32)
    s = jax.nn.softmax(s, axis=0)
    o_ref[...] = _lrelu(e * s)


def _attn(ea, eb, w):
    return pl.pallas_call(
        _attn_body,
        in_specs=[pl.BlockSpec((NODES, D), lambda: (0, 0)),
                  pl.BlockSpec((NODES, D), lambda: (0, 0)),
                  pl.BlockSpec((D, 1), lambda: (0, 0))],
        out_specs=pl.BlockSpec((NODES, D), lambda: (0, 0)),
        out_shape=jax.ShapeDtypeStruct((NODES, D), jnp.float32),
    )(ea, eb, w)


def _final_body(ea_ref, eb_ref, w_ref, modal_ref, g1_ref, o_ref):
    e = ea_ref[...] + eb_ref[...]
    s = jnp.dot(e, w_ref[...], preferred_element_type=jnp.float32)
    s = jax.nn.softmax(s, axis=0)
    g2 = _lrelu(e * s)
    m = modal_ref[...]
    n = jnp.sqrt(jnp.sum(m * m, axis=1, keepdims=True))
    o_ref[...] = m + g1_ref[...] + g2 + RIS_LAM * (m / jnp.maximum(n, 1e-12))


def _final(ea, eb, w, modal, g1):
    return pl.pallas_call(
        _final_body,
        in_specs=[pl.BlockSpec((NODES, D), lambda: (0, 0)),
                  pl.BlockSpec((NODES, D), lambda: (0, 0)),
                  pl.BlockSpec((D, 1), lambda: (0, 0)),
                  pl.BlockSpec((NODES, D), lambda: (0, 0)),
                  pl.BlockSpec((NODES, D), lambda: (0, 0))],
        out_specs=pl.BlockSpec((NODES, D), lambda: (0, 0)),
        out_shape=jax.ShapeDtypeStruct((NODES, D), jnp.float32),
    )(ea, eb, w, modal, g1)


def kernel(adj_idx, adj_vals, image_adj_idx, image_adj_vals, text_adj_idx,
           text_adj_vals, image_embedding, text_embedding, uEmbeds, iEmbeds,
           image_trans, text_trans, modal_weight, att_w0, att_w1):
    img_n = _feats(image_embedding, image_trans)
    txt_n = _feats(text_embedding, text_trans)

    base = jnp.concatenate([uEmbeds, iEmbeds], axis=0)
    x4 = jnp.concatenate([uEmbeds, txt_n], axis=0)

    s14 = _spmm4(image_adj_idx, image_adj_vals, adj_idx, adj_vals,
                 text_adj_idx, text_adj_vals, base, x4)
    s1, s2, s3, s4 = s14[0], s14[1], s14[2], s14[3]

    x5 = jnp.concatenate([s4[:USER_N], iEmbeds], axis=0)
    s5 = _spmm1(adj_idx, adj_vals, x5)

    imgbase = jnp.concatenate([uEmbeds, img_n], axis=0)
    modal = _combine(modal_weight, imgbase, s1, s2, s3, s4, s5[0], s5[1])

    e1 = _spmm1(adj_idx, adj_vals, modal)
    g1 = _attn(e1[0], e1[1], att_w0)
    e2 = _spmm1(adj_idx, adj_vals, g1)
    out = _final(e2[0], e2[1], att_w1, modal, g1)
    return (out[:USER_N], out[USER_N:])
